# baseline (speedup 1.0000x reference)
import jax
import jax.numpy as jnp
from jax.experimental import pallas as pl
from jax.experimental.pallas import tpu as pltpu

_BATCH = 4


def _copy_kernel(tbl, out, buf, in_sem, out_sem):
    pltpu.make_async_copy(tbl, buf, in_sem).start()
    pltpu.make_async_copy(tbl, buf, in_sem).wait()
    for b in range(_BATCH):
        pltpu.make_async_copy(buf, out.at[b], out_sem.at[b]).start()
    for b in range(_BATCH):
        pltpu.make_async_copy(buf, out.at[b], out_sem.at[b]).wait()


def kernel(inputs, table):
    del inputs
    num_rows, dim = table.shape
    out = pl.pallas_call(
        _copy_kernel,
        in_specs=[pl.BlockSpec(memory_space=pl.ANY)],
        out_specs=pl.BlockSpec(memory_space=pl.ANY),
        out_shape=jax.ShapeDtypeStruct((_BATCH, num_rows, dim), table.dtype),
        scratch_shapes=[
            pltpu.VMEM((num_rows, dim), table.dtype),
            pltpu.SemaphoreType.DMA,
            pltpu.SemaphoreType.DMA((_BATCH,)),
        ],
    )(table)
    return out
